# baseline (speedup 1.0000x reference)
import functools

import jax
import jax.numpy as jnp
from jax import lax
from jax.experimental import pallas as pl
from jax.experimental.pallas import tpu as pltpu
from jax.experimental.pallas import tpu_sc as plsc

NC = 2
NS = 16
LANES = 16
F = 128
BN = 400


def _sc_mesh():
    return plsc.VectorSubcoreMesh(core_axis_name="c", subcore_axis_name="s")


_CH = 128
_CHP = 64


def _degree_body(n_pad, nchd, dst_hbm, ones_hbm, z_hbm, out_hbm, didx, onesv,
                 acc, sem0, sem1):
    c = lax.axis_index("c")
    s = lax.axis_index("s")
    wid = c * NS + s
    npw = n_pad // NS
    sems = (sem0, sem1)

    pltpu.sync_copy(ones_hbm, onesv)
    pltpu.sync_copy(dst_hbm.at[wid], didx)
    pltpu.sync_copy(z_hbm.at[pl.ds(s * npw, npw)], acc.at[pl.ds(s * npw, npw)])
    plsc.subcore_barrier()

    def gbody(g, carry):
        for b in range(2):
            k = g * 2 + b

            @pl.when(k >= 2)
            def _():
                pltpu.make_async_copy(onesv, acc.at[didx.at[k - 2]],
                                      sems[b]).wait()

            pltpu.async_copy(onesv, acc.at[didx.at[k]], sems[b], add=True)
        return carry

    lax.fori_loop(0, nchd // 2, gbody, 0)
    for b in range(2):
        pltpu.make_async_copy(onesv, acc.at[didx.at[nchd - 2 + b]],
                              sems[b]).wait()

    plsc.subcore_barrier()
    pltpu.sync_copy(acc.at[pl.ds(s * npw, npw)],
                    out_hbm.at[pl.ds(c * n_pad + s * npw, npw)])


def _sc_degree(dst3d, ones, zeros2d, n_pad):
    nchd = dst3d.shape[1]
    kfn = pl.kernel(
        functools.partial(_degree_body, n_pad, nchd),
        out_type=jax.ShapeDtypeStruct((NC * n_pad, F), jnp.float32),
        mesh=_sc_mesh(),
        scratch_types=[
            pltpu.VMEM((nchd, _CH), jnp.int32),
            pltpu.VMEM((_CH, F), jnp.float32),
            pltpu.VMEM_SHARED((n_pad, F), jnp.float32),
            pltpu.SemaphoreType.DMA,
            pltpu.SemaphoreType.DMA,
        ],
    )
    return kfn(dst3d, ones, zeros2d)


_RING = 3


def _prop_body(n_pad, nch, tlo_hbm, thi_hbm, src_hbm, dst_hbm, z_hbm, olo_hbm,
               ohi_hbm, didx, sidx, rows0, rows1, rows2, acc,
               gs0, gs1, gs2, ss0, ss1, ss2):
    c = lax.axis_index("c")
    s = lax.axis_index("s")
    rpt = n_pad // NS
    rows = (rows0, rows1, rows2)
    gsem = (gs0, gs1, gs2)
    ssem = (ss0, ss1, ss2)

    pltpu.sync_copy(dst_hbm.at[s], didx)
    pltpu.sync_copy(src_hbm.at[s], sidx)
    pltpu.sync_copy(z_hbm.at[pl.ds(s * rpt, rpt)], acc.at[pl.ds(s * rpt, rpt)])
    plsc.subcore_barrier()

    def svec(tbl, k):
        return tbl.at[k >> 1, pl.ds((k & 1) * _CHP, _CHP)]

    def edge_loop(table):
        def gbody(g, carry):
            for b in range(_RING):
                k = g * _RING + b
                bp = (b + _RING - 1) % _RING

                @pl.when(k >= _RING)
                def _():
                    pltpu.make_async_copy(rows[b], acc.at[svec(didx, k - _RING)],
                                          ssem[b]).wait()

                pltpu.async_copy(table.at[svec(sidx, k)], rows[b], gsem[b])

                @pl.when(k >= 1)
                def _():
                    pltpu.make_async_copy(table.at[svec(sidx, k - 1)], rows[bp],
                                          gsem[bp]).wait()
                    pltpu.async_copy(rows[bp], acc.at[svec(didx, k - 1)],
                                     ssem[bp], add=True)
            return carry

        lax.fori_loop(0, nch // _RING, gbody, 0)
        pltpu.make_async_copy(table.at[svec(sidx, nch - 1)], rows[_RING - 1],
                              gsem[_RING - 1]).wait()
        pltpu.async_copy(rows[_RING - 1], acc.at[svec(didx, nch - 1)],
                         ssem[_RING - 1], add=True)
        for b in range(_RING):
            pltpu.make_async_copy(rows[b], acc.at[svec(didx, b)],
                                  ssem[b]).wait()

    @pl.when(c == 0)
    def _():
        edge_loop(tlo_hbm)

    @pl.when(c == 1)
    def _():
        edge_loop(thi_hbm)

    plsc.subcore_barrier()

    @pl.when(c == 0)
    def _():
        pltpu.sync_copy(acc.at[pl.ds(s * rpt, rpt)],
                        olo_hbm.at[pl.ds(s * rpt, rpt)])

    @pl.when(c == 1)
    def _():
        pltpu.sync_copy(acc.at[pl.ds(s * rpt, rpt)],
                        ohi_hbm.at[pl.ds(s * rpt, rpt)])


def _sc_prop(tlo, thi, src4, dst3, zeros, n_pad):
    nch2 = src4.shape[1]
    kfn = pl.kernel(
        functools.partial(_prop_body, n_pad, 2 * nch2),
        out_type=(jax.ShapeDtypeStruct((n_pad, F), jnp.float32),
                  jax.ShapeDtypeStruct((n_pad, F), jnp.float32)),
        mesh=_sc_mesh(),
        scratch_types=[
            pltpu.VMEM((nch2, _CH), jnp.int32),
            pltpu.VMEM((nch2, _CH), jnp.int32),
            pltpu.VMEM((_CHP, F), jnp.float32),
            pltpu.VMEM((_CHP, F), jnp.float32),
            pltpu.VMEM((_CHP, F), jnp.float32),
            pltpu.VMEM_SHARED((n_pad, F), jnp.float32),
        ] + [pltpu.SemaphoreType.DMA] * 6,
    )
    return kfn(tlo, thi, src4, dst3, zeros)


def _prep_body(deg_ref, x_ref, t0lo_ref, t0hi_ref, norm_ref, norm2_ref):
    d = jnp.maximum(deg_ref[...], 1.0)
    norm = lax.rsqrt(d)
    xb = x_ref[...]
    t0lo_ref[...] = xb[:, :F] * norm
    t0hi_ref[...] = xb[:, F:] * norm
    norm_ref[...] = norm
    norm2_ref[...] = 1.0 / d


def _tc_prep(deg2, x):
    n = x.shape[0]
    g = n // BN
    return pl.pallas_call(
        _prep_body,
        grid=(g,),
        in_specs=[
            pl.BlockSpec((BN, 1), lambda i: (i, 0)),
            pl.BlockSpec((BN, 2 * F), lambda i: (i, 0)),
        ],
        out_specs=[
            pl.BlockSpec((BN, F), lambda i: (i, 0)),
            pl.BlockSpec((BN, F), lambda i: (i, 0)),
            pl.BlockSpec((BN, 1), lambda i: (i, 0)),
            pl.BlockSpec((BN, 1), lambda i: (i, 0)),
        ],
        out_shape=[
            jax.ShapeDtypeStruct((n, F), jnp.float32),
            jax.ShapeDtypeStruct((n, F), jnp.float32),
            jax.ShapeDtypeStruct((n, 1), jnp.float32),
            jax.ShapeDtypeStruct((n, 1), jnp.float32),
        ],
    )(deg2, x)


def _scale_body(alo_ref, ahi_ref, s_ref, olo_ref, ohi_ref):
    sb = s_ref[...]
    olo_ref[...] = alo_ref[...] * sb
    ohi_ref[...] = ahi_ref[...] * sb


def _tc_scale(alo, ahi, s):
    n = alo.shape[0]
    g = n // BN
    return pl.pallas_call(
        _scale_body,
        grid=(g,),
        in_specs=[
            pl.BlockSpec((BN, F), lambda i: (i, 0)),
            pl.BlockSpec((BN, F), lambda i: (i, 0)),
            pl.BlockSpec((BN, 1), lambda i: (i, 0)),
        ],
        out_specs=[
            pl.BlockSpec((BN, F), lambda i: (i, 0)),
            pl.BlockSpec((BN, F), lambda i: (i, 0)),
        ],
        out_shape=[
            jax.ShapeDtypeStruct((n, F), jnp.float32),
            jax.ShapeDtypeStruct((n, F), jnp.float32),
        ],
    )(alo, ahi, s)


def _scale_add_body(alo_ref, ahi_ref, s_ref, blo_ref, bhi_ref, olo_ref,
                    ohi_ref):
    sb = s_ref[...]
    olo_ref[...] = alo_ref[...] * sb + blo_ref[...]
    ohi_ref[...] = ahi_ref[...] * sb + bhi_ref[...]


def _tc_scale_add(alo, ahi, s, blo, bhi):
    n = alo.shape[0]
    g = n // BN
    return pl.pallas_call(
        _scale_add_body,
        grid=(g,),
        in_specs=[pl.BlockSpec((BN, F), lambda i: (i, 0)),
                  pl.BlockSpec((BN, F), lambda i: (i, 0)),
                  pl.BlockSpec((BN, 1), lambda i: (i, 0)),
                  pl.BlockSpec((BN, F), lambda i: (i, 0)),
                  pl.BlockSpec((BN, F), lambda i: (i, 0))],
        out_specs=[pl.BlockSpec((BN, F), lambda i: (i, 0)),
                   pl.BlockSpec((BN, F), lambda i: (i, 0))],
        out_shape=[jax.ShapeDtypeStruct((n, F), jnp.float32),
                   jax.ShapeDtypeStruct((n, F), jnp.float32)],
    )(alo, ahi, s, blo, bhi)


def _mid_body(x_ref, p1lo_ref, p1hi_ref, p2lo_ref, p2hi_ref, n_ref, W1_ref,
              b1_ref, W2r_ref, v1_ref, v2nlo_ref, v2nhi_ref, t2lo_ref,
              t2hi_ref):
    nb = n_ref[...]
    cat = jnp.concatenate(
        [x_ref[...],
         p1lo_ref[...] * nb, p1hi_ref[...] * nb,
         p2lo_ref[...] * nb, p2hi_ref[...] * nb], axis=1)
    h1 = jnp.dot(cat, W1_ref[...], preferred_element_type=jnp.float32)
    h1 = jnp.maximum(h1 + b1_ref[...], 0.0)
    v = jnp.dot(h1, W2r_ref[...], preferred_element_type=jnp.float32)
    v1_ref[...] = v[:, :2 * F]
    v2nlo_ref[...] = v[:, 2 * F:3 * F] * nb
    v2nhi_ref[...] = v[:, 3 * F:4 * F] * nb
    t2lo_ref[...] = v[:, 4 * F:5 * F] * nb
    t2hi_ref[...] = v[:, 5 * F:6 * F] * nb


def _tc_mid(x, p1lo, p1hi, p2lo, p2hi, norm, W1, b1, W2r):
    n = x.shape[0]
    g = n // BN
    in_dim = x.shape[1]
    hid = W1.shape[1]
    return pl.pallas_call(
        _mid_body,
        grid=(g,),
        in_specs=[
            pl.BlockSpec((BN, in_dim), lambda i: (i, 0)),
            pl.BlockSpec((BN, F), lambda i: (i, 0)),
            pl.BlockSpec((BN, F), lambda i: (i, 0)),
            pl.BlockSpec((BN, F), lambda i: (i, 0)),
            pl.BlockSpec((BN, F), lambda i: (i, 0)),
            pl.BlockSpec((BN, 1), lambda i: (i, 0)),
            pl.BlockSpec(W1.shape, lambda i: (0, 0)),
            pl.BlockSpec((1, hid), lambda i: (0, 0)),
            pl.BlockSpec(W2r.shape, lambda i: (0, 0)),
        ],
        out_specs=[
            pl.BlockSpec((BN, 2 * F), lambda i: (i, 0)),
            pl.BlockSpec((BN, F), lambda i: (i, 0)),
            pl.BlockSpec((BN, F), lambda i: (i, 0)),
            pl.BlockSpec((BN, F), lambda i: (i, 0)),
            pl.BlockSpec((BN, F), lambda i: (i, 0)),
        ],
        out_shape=[
            jax.ShapeDtypeStruct((n, 2 * F), jnp.float32),
            jax.ShapeDtypeStruct((n, F), jnp.float32),
            jax.ShapeDtypeStruct((n, F), jnp.float32),
            jax.ShapeDtypeStruct((n, F), jnp.float32),
            jax.ShapeDtypeStruct((n, F), jnp.float32),
        ],
    )(x, p1lo, p1hi, p2lo, p2hi, norm, W1, b1, W2r)


def _head_body(g, n, v1_ref, qlo_ref, qhi_ref, n_ref, b2_ref, Wc_ref, bc_ref,
               y_ref, acc_ref):
    i = pl.program_id(0)
    nb = n_ref[...]
    h2 = jnp.concatenate([qlo_ref[...], qhi_ref[...]], axis=1) * nb
    h2 = jnp.maximum(h2 + v1_ref[...] + b2_ref[...], 0.0)
    part = jnp.sum(h2, axis=0, keepdims=True)

    @pl.when(i == 0)
    def _():
        acc_ref[...] = part

    @pl.when(i > 0)
    def _():
        acc_ref[...] = acc_ref[...] + part

    @pl.when(i == g - 1)
    def _():
        hg = acc_ref[...] * (1.0 / n)
        y_ref[...] = (jnp.dot(hg, Wc_ref[...],
                              preferred_element_type=jnp.float32)
                      + bc_ref[...])


def _tc_head(v1, qlo, qhi, norm, b2, Wc, bc):
    n = v1.shape[0]
    g = n // BN
    ncls = Wc.shape[1]
    return pl.pallas_call(
        functools.partial(_head_body, g, float(n)),
        grid=(g,),
        in_specs=[
            pl.BlockSpec((BN, 2 * F), lambda i: (i, 0)),
            pl.BlockSpec((BN, F), lambda i: (i, 0)),
            pl.BlockSpec((BN, F), lambda i: (i, 0)),
            pl.BlockSpec((BN, 1), lambda i: (i, 0)),
            pl.BlockSpec((1, 2 * F), lambda i: (0, 0)),
            pl.BlockSpec(Wc.shape, lambda i: (0, 0)),
            pl.BlockSpec((1, ncls), lambda i: (0, 0)),
        ],
        out_specs=pl.BlockSpec((1, ncls), lambda i: (0, 0)),
        out_shape=jax.ShapeDtypeStruct((1, ncls), jnp.float32),
        scratch_shapes=[pltpu.VMEM((1, 2 * F), jnp.float32)],
    )(v1, qlo, qhi, norm, b2, Wc, bc)


def kernel(x, edge_index, W1, b1, W2, b2, Wc, bc):
    n, in_dim = x.shape
    e = edge_index.shape[1]
    hid = W1.shape[1]
    out2 = W2.shape[1]
    assert in_dim == 2 * F and n % BN == 0 and n % NS == 0

    src = edge_index[0]
    dst = edge_index[1]
    n_pad = ((n + NS * LANES - 1) // (NS * LANES)) * NS * LANES
    assert n_pad > n
    zeros = jnp.zeros((n_pad, F), jnp.float32)

    nch2 = -(-e // (NS * _CH))
    while (2 * nch2) % _RING:
        nch2 += 1
    ep = NS * nch2 * _CH
    ar = jnp.arange(ep - e, dtype=jnp.int32)
    src_p = jnp.concatenate([src, ar % n])
    dst_p = jnp.concatenate([dst, n + ar % (n_pad - n)])
    src4 = src_p.reshape(NS, nch2, _CH)
    dst3 = dst_p.reshape(NS, nch2, _CH)

    nchd = -(-e // (NC * NS * _CH))
    nchd = ((nchd + 1) // 2) * 2
    epd = NC * NS * nchd * _CH
    ard = jnp.arange(epd - e, dtype=jnp.int32)
    dst_pd = jnp.concatenate([dst, n + ard % (n_pad - n)])
    dst3d = dst_pd.reshape(NC * NS, nchd, _CH)

    W2r = jnp.concatenate([W2[:hid], W2[hid:2 * hid], W2[2 * hid:]], axis=1)

    deg2 = _sc_degree(dst3d, jnp.ones((_CH, F), jnp.float32), zeros, n_pad)
    degcol = (deg2[:n, 0] + deg2[n_pad:n_pad + n, 0]).reshape(n, 1)
    t0lo, t0hi, norm, norm2 = _tc_prep(degcol, x)
    p1lo, p1hi = _sc_prop(t0lo, t0hi, src4, dst3, zeros, n_pad)
    t1lo, t1hi = _tc_scale(p1lo, p1hi, norm2)
    p2lo, p2hi = _sc_prop(t1lo, t1hi, src4, dst3, zeros, n_pad)
    v1, v2nlo, v2nhi, t2lo, t2hi = _tc_mid(
        x, p1lo, p1hi, p2lo, p2hi, norm, W1, b1.reshape(1, hid), W2r)
    q1lo, q1hi = _sc_prop(t2lo, t2hi, src4, dst3, zeros, n_pad)
    t3lo, t3hi = _tc_scale_add(q1lo, q1hi, norm2, v2nlo, v2nhi)
    q2lo, q2hi = _sc_prop(t3lo, t3hi, src4, dst3, zeros, n_pad)
    y = _tc_head(v1, q2lo, q2hi, norm, b2.reshape(1, out2), Wc,
                 bc.reshape(1, -1))
    return y


# speedup vs baseline: 1.0005x; 1.0005x over previous
import functools

import jax
import jax.numpy as jnp
from jax import lax
from jax.experimental import pallas as pl
from jax.experimental.pallas import tpu as pltpu
from jax.experimental.pallas import tpu_sc as plsc

NC = 2
NS = 16
LANES = 16
F = 128
BN = 400


def _sc_mesh():
    return plsc.VectorSubcoreMesh(core_axis_name="c", subcore_axis_name="s")


_CH = 128
_CHP = 64


def _degree_body(n_pad, nchd, dst_hbm, ones_hbm, z_hbm, out_hbm, didx, onesv,
                 acc, sem0, sem1):
    c = lax.axis_index("c")
    s = lax.axis_index("s")
    wid = c * NS + s
    npw = n_pad // NS
    sems = (sem0, sem1)

    pltpu.sync_copy(ones_hbm, onesv)
    pltpu.sync_copy(dst_hbm.at[wid], didx)
    pltpu.sync_copy(z_hbm.at[pl.ds(s * npw, npw)], acc.at[pl.ds(s * npw, npw)])
    plsc.subcore_barrier()

    def gbody(g, carry):
        for b in range(2):
            k = g * 2 + b

            @pl.when(k >= 2)
            def _():
                pltpu.make_async_copy(onesv, acc.at[didx.at[k - 2]],
                                      sems[b]).wait()

            pltpu.async_copy(onesv, acc.at[didx.at[k]], sems[b], add=True)
        return carry

    lax.fori_loop(0, nchd // 2, gbody, 0)
    for b in range(2):
        pltpu.make_async_copy(onesv, acc.at[didx.at[nchd - 2 + b]],
                              sems[b]).wait()

    plsc.subcore_barrier()
    pltpu.sync_copy(acc.at[pl.ds(s * npw, npw)],
                    out_hbm.at[pl.ds(c * n_pad + s * npw, npw)])


def _sc_degree(dst3d, ones, zeros2d, n_pad):
    nchd = dst3d.shape[1]
    kfn = pl.kernel(
        functools.partial(_degree_body, n_pad, nchd),
        out_type=jax.ShapeDtypeStruct((NC * n_pad, F), jnp.float32),
        mesh=_sc_mesh(),
        scratch_types=[
            pltpu.VMEM((nchd, _CH), jnp.int32),
            pltpu.VMEM((_CH, F), jnp.float32),
            pltpu.VMEM_SHARED((n_pad, F), jnp.float32),
            pltpu.SemaphoreType.DMA,
            pltpu.SemaphoreType.DMA,
        ],
    )
    return kfn(dst3d, ones, zeros2d)


_RING = 3


def _prop_body(n_pad, nch, tlo_hbm, thi_hbm, src_hbm, dst_hbm, z_hbm, olo_hbm,
               ohi_hbm, didx, sidx, rows0, rows1, rows2, acc,
               gs0, gs1, gs2, ss0, ss1, ss2):
    c = lax.axis_index("c")
    s = lax.axis_index("s")
    rpt = n_pad // NS
    rows = (rows0, rows1, rows2)
    gsem = (gs0, gs1, gs2)
    ssem = (ss0, ss1, ss2)

    pltpu.sync_copy(dst_hbm.at[s], didx)
    pltpu.sync_copy(src_hbm.at[s], sidx)
    pltpu.sync_copy(z_hbm.at[pl.ds(s * rpt, rpt)], acc.at[pl.ds(s * rpt, rpt)])
    plsc.subcore_barrier()

    def svec(tbl, k):
        return tbl.at[k >> 1, pl.ds((k & 1) * _CHP, _CHP)]

    def edge_loop(table):
        def gbody(g, carry):
            for b in range(_RING):
                k = g * _RING + b
                bp = (b + _RING - 1) % _RING

                @pl.when(k >= _RING)
                def _():
                    pltpu.make_async_copy(rows[b], acc.at[svec(didx, k - _RING)],
                                          ssem[b]).wait()

                pltpu.async_copy(table.at[svec(sidx, k)], rows[b], gsem[b])

                @pl.when(k >= 1)
                def _():
                    pltpu.make_async_copy(table.at[svec(sidx, k - 1)], rows[bp],
                                          gsem[bp]).wait()
                    pltpu.async_copy(rows[bp], acc.at[svec(didx, k - 1)],
                                     ssem[bp], add=True)
            return carry

        lax.fori_loop(0, nch // _RING, gbody, 0)
        pltpu.make_async_copy(table.at[svec(sidx, nch - 1)], rows[_RING - 1],
                              gsem[_RING - 1]).wait()
        pltpu.async_copy(rows[_RING - 1], acc.at[svec(didx, nch - 1)],
                         ssem[_RING - 1], add=True)
        for b in range(_RING):
            pltpu.make_async_copy(rows[b], acc.at[svec(didx, b)],
                                  ssem[b]).wait()

    @pl.when(c == 0)
    def _():
        edge_loop(tlo_hbm)

    @pl.when(c == 1)
    def _():
        edge_loop(thi_hbm)

    plsc.subcore_barrier()

    @pl.when(c == 0)
    def _():
        pltpu.sync_copy(acc.at[pl.ds(s * rpt, rpt)],
                        olo_hbm.at[pl.ds(s * rpt, rpt)])

    @pl.when(c == 1)
    def _():
        pltpu.sync_copy(acc.at[pl.ds(s * rpt, rpt)],
                        ohi_hbm.at[pl.ds(s * rpt, rpt)])


def _sc_prop(tlo, thi, src4, dst3, zeros, n_pad):
    nch2 = src4.shape[1]
    kfn = pl.kernel(
        functools.partial(_prop_body, n_pad, 2 * nch2),
        out_type=(jax.ShapeDtypeStruct((n_pad, F), jnp.float32),
                  jax.ShapeDtypeStruct((n_pad, F), jnp.float32)),
        mesh=_sc_mesh(),
        scratch_types=[
            pltpu.VMEM((nch2, _CH), jnp.int32),
            pltpu.VMEM((nch2, _CH), jnp.int32),
            pltpu.VMEM((_CHP, F), jnp.float32),
            pltpu.VMEM((_CHP, F), jnp.float32),
            pltpu.VMEM((_CHP, F), jnp.float32),
            pltpu.VMEM_SHARED((n_pad, F), jnp.float32),
        ] + [pltpu.SemaphoreType.DMA] * 6,
    )
    return kfn(tlo, thi, src4, dst3, zeros)


def _prep_body(deg_ref, x_ref, t0lo_ref, t0hi_ref, norm_ref, norm2_ref):
    d = jnp.maximum(deg_ref[...], 1.0)
    norm = lax.rsqrt(d)
    xb = x_ref[...]
    t0lo_ref[...] = xb[:, :F] * norm
    t0hi_ref[...] = xb[:, F:] * norm
    norm_ref[...] = norm
    norm2_ref[...] = 1.0 / d


def _tc_prep(deg2, x):
    n = x.shape[0]
    g = n // BN
    return pl.pallas_call(
        _prep_body,
        grid=(g,),
        in_specs=[
            pl.BlockSpec((BN, 1), lambda i: (i, 0)),
            pl.BlockSpec((BN, 2 * F), lambda i: (i, 0)),
        ],
        out_specs=[
            pl.BlockSpec((BN, F), lambda i: (i, 0)),
            pl.BlockSpec((BN, F), lambda i: (i, 0)),
            pl.BlockSpec((BN, 1), lambda i: (i, 0)),
            pl.BlockSpec((BN, 1), lambda i: (i, 0)),
        ],
        out_shape=[
            jax.ShapeDtypeStruct((n, F), jnp.float32),
            jax.ShapeDtypeStruct((n, F), jnp.float32),
            jax.ShapeDtypeStruct((n, 1), jnp.float32),
            jax.ShapeDtypeStruct((n, 1), jnp.float32),
        ],
    )(deg2, x)


def _scale_body(alo_ref, ahi_ref, s_ref, olo_ref, ohi_ref):
    sb = s_ref[...]
    olo_ref[...] = alo_ref[...] * sb
    ohi_ref[...] = ahi_ref[...] * sb


def _tc_scale(alo, ahi, s):
    n = alo.shape[0]
    g = n // BN
    return pl.pallas_call(
        _scale_body,
        grid=(g,),
        in_specs=[
            pl.BlockSpec((BN, F), lambda i: (i, 0)),
            pl.BlockSpec((BN, F), lambda i: (i, 0)),
            pl.BlockSpec((BN, 1), lambda i: (i, 0)),
        ],
        out_specs=[
            pl.BlockSpec((BN, F), lambda i: (i, 0)),
            pl.BlockSpec((BN, F), lambda i: (i, 0)),
        ],
        out_shape=[
            jax.ShapeDtypeStruct((n, F), jnp.float32),
            jax.ShapeDtypeStruct((n, F), jnp.float32),
        ],
    )(alo, ahi, s)


def _scale_add_body(alo_ref, ahi_ref, s_ref, blo_ref, bhi_ref, olo_ref,
                    ohi_ref):
    sb = s_ref[...]
    olo_ref[...] = alo_ref[...] * sb + blo_ref[...]
    ohi_ref[...] = ahi_ref[...] * sb + bhi_ref[...]


def _tc_scale_add(alo, ahi, s, blo, bhi):
    n = alo.shape[0]
    g = n // BN
    return pl.pallas_call(
        _scale_add_body,
        grid=(g,),
        in_specs=[pl.BlockSpec((BN, F), lambda i: (i, 0)),
                  pl.BlockSpec((BN, F), lambda i: (i, 0)),
                  pl.BlockSpec((BN, 1), lambda i: (i, 0)),
                  pl.BlockSpec((BN, F), lambda i: (i, 0)),
                  pl.BlockSpec((BN, F), lambda i: (i, 0))],
        out_specs=[pl.BlockSpec((BN, F), lambda i: (i, 0)),
                   pl.BlockSpec((BN, F), lambda i: (i, 0))],
        out_shape=[jax.ShapeDtypeStruct((n, F), jnp.float32),
                   jax.ShapeDtypeStruct((n, F), jnp.float32)],
    )(alo, ahi, s, blo, bhi)


def _mid_a_body(x_ref, p1lo_ref, p1hi_ref, p2lo_ref, p2hi_ref, n_ref, W1_ref,
                b1_ref, W2c_ref, h1_ref, t2lo_ref, t2hi_ref):
    nb = n_ref[...]
    cat = jnp.concatenate(
        [x_ref[...],
         p1lo_ref[...] * nb, p1hi_ref[...] * nb,
         p2lo_ref[...] * nb, p2hi_ref[...] * nb], axis=1)
    h1 = jnp.dot(cat, W1_ref[...], preferred_element_type=jnp.float32)
    h1 = jnp.maximum(h1 + b1_ref[...], 0.0)
    h1_ref[...] = h1
    v = jnp.dot(h1, W2c_ref[...], preferred_element_type=jnp.float32)
    t2lo_ref[...] = v[:, :F] * nb
    t2hi_ref[...] = v[:, F:] * nb


def _tc_mid_a(x, p1lo, p1hi, p2lo, p2hi, norm, W1, b1, W2c):
    n = x.shape[0]
    g = n // BN
    in_dim = x.shape[1]
    hid = W1.shape[1]
    return pl.pallas_call(
        _mid_a_body,
        grid=(g,),
        in_specs=[
            pl.BlockSpec((BN, in_dim), lambda i: (i, 0)),
            pl.BlockSpec((BN, F), lambda i: (i, 0)),
            pl.BlockSpec((BN, F), lambda i: (i, 0)),
            pl.BlockSpec((BN, F), lambda i: (i, 0)),
            pl.BlockSpec((BN, F), lambda i: (i, 0)),
            pl.BlockSpec((BN, 1), lambda i: (i, 0)),
            pl.BlockSpec(W1.shape, lambda i: (0, 0)),
            pl.BlockSpec((1, hid), lambda i: (0, 0)),
            pl.BlockSpec(W2c.shape, lambda i: (0, 0)),
        ],
        out_specs=[
            pl.BlockSpec((BN, hid), lambda i: (i, 0)),
            pl.BlockSpec((BN, F), lambda i: (i, 0)),
            pl.BlockSpec((BN, F), lambda i: (i, 0)),
        ],
        out_shape=[
            jax.ShapeDtypeStruct((n, hid), jnp.float32),
            jax.ShapeDtypeStruct((n, F), jnp.float32),
            jax.ShapeDtypeStruct((n, F), jnp.float32),
        ],
    )(x, p1lo, p1hi, p2lo, p2hi, norm, W1, b1, W2c)


def _mid_b_body(h1_ref, n_ref, W2ab_ref, v1_ref, v2nlo_ref, v2nhi_ref):
    nb = n_ref[...]
    v = jnp.dot(h1_ref[...], W2ab_ref[...], preferred_element_type=jnp.float32)
    v1_ref[...] = v[:, :2 * F]
    v2nlo_ref[...] = v[:, 2 * F:3 * F] * nb
    v2nhi_ref[...] = v[:, 3 * F:4 * F] * nb


def _tc_mid_b(h1, norm, W2ab):
    n = h1.shape[0]
    g = n // BN
    hid = h1.shape[1]
    return pl.pallas_call(
        _mid_b_body,
        grid=(g,),
        in_specs=[
            pl.BlockSpec((BN, hid), lambda i: (i, 0)),
            pl.BlockSpec((BN, 1), lambda i: (i, 0)),
            pl.BlockSpec(W2ab.shape, lambda i: (0, 0)),
        ],
        out_specs=[
            pl.BlockSpec((BN, 2 * F), lambda i: (i, 0)),
            pl.BlockSpec((BN, F), lambda i: (i, 0)),
            pl.BlockSpec((BN, F), lambda i: (i, 0)),
        ],
        out_shape=[
            jax.ShapeDtypeStruct((n, 2 * F), jnp.float32),
            jax.ShapeDtypeStruct((n, F), jnp.float32),
            jax.ShapeDtypeStruct((n, F), jnp.float32),
        ],
    )(h1, norm, W2ab)


def _head_body(g, n, v1_ref, qlo_ref, qhi_ref, n_ref, b2_ref, Wc_ref, bc_ref,
               y_ref, acc_ref):
    i = pl.program_id(0)
    nb = n_ref[...]
    h2 = jnp.concatenate([qlo_ref[...], qhi_ref[...]], axis=1) * nb
    h2 = jnp.maximum(h2 + v1_ref[...] + b2_ref[...], 0.0)
    part = jnp.sum(h2, axis=0, keepdims=True)

    @pl.when(i == 0)
    def _():
        acc_ref[...] = part

    @pl.when(i > 0)
    def _():
        acc_ref[...] = acc_ref[...] + part

    @pl.when(i == g - 1)
    def _():
        hg = acc_ref[...] * (1.0 / n)
        y_ref[...] = (jnp.dot(hg, Wc_ref[...],
                              preferred_element_type=jnp.float32)
                      + bc_ref[...])


def _tc_head(v1, qlo, qhi, norm, b2, Wc, bc):
    n = v1.shape[0]
    g = n // BN
    ncls = Wc.shape[1]
    return pl.pallas_call(
        functools.partial(_head_body, g, float(n)),
        grid=(g,),
        in_specs=[
            pl.BlockSpec((BN, 2 * F), lambda i: (i, 0)),
            pl.BlockSpec((BN, F), lambda i: (i, 0)),
            pl.BlockSpec((BN, F), lambda i: (i, 0)),
            pl.BlockSpec((BN, 1), lambda i: (i, 0)),
            pl.BlockSpec((1, 2 * F), lambda i: (0, 0)),
            pl.BlockSpec(Wc.shape, lambda i: (0, 0)),
            pl.BlockSpec((1, ncls), lambda i: (0, 0)),
        ],
        out_specs=pl.BlockSpec((1, ncls), lambda i: (0, 0)),
        out_shape=jax.ShapeDtypeStruct((1, ncls), jnp.float32),
        scratch_shapes=[pltpu.VMEM((1, 2 * F), jnp.float32)],
    )(v1, qlo, qhi, norm, b2, Wc, bc)


def kernel(x, edge_index, W1, b1, W2, b2, Wc, bc):
    n, in_dim = x.shape
    e = edge_index.shape[1]
    hid = W1.shape[1]
    out2 = W2.shape[1]
    assert in_dim == 2 * F and n % BN == 0 and n % NS == 0

    src = edge_index[0]
    dst = edge_index[1]
    n_pad = ((n + NS * LANES - 1) // (NS * LANES)) * NS * LANES
    assert n_pad > n
    zeros = jnp.zeros((n_pad, F), jnp.float32)

    nch2 = -(-e // (NS * _CH))
    while (2 * nch2) % _RING:
        nch2 += 1
    ep = NS * nch2 * _CH
    ar = jnp.arange(ep - e, dtype=jnp.int32)
    src_p = jnp.concatenate([src, ar % n])
    dst_p = jnp.concatenate([dst, n + ar % (n_pad - n)])
    src4 = src_p.reshape(NS, nch2, _CH)
    dst3 = dst_p.reshape(NS, nch2, _CH)

    nchd = -(-e // (NC * NS * _CH))
    nchd = ((nchd + 1) // 2) * 2
    epd = NC * NS * nchd * _CH
    ard = jnp.arange(epd - e, dtype=jnp.int32)
    dst_pd = jnp.concatenate([dst, n + ard % (n_pad - n)])
    dst3d = dst_pd.reshape(NC * NS, nchd, _CH)

    W2c = W2[2 * hid:]
    W2ab = jnp.concatenate([W2[:hid], W2[hid:2 * hid]], axis=1)

    deg2 = _sc_degree(dst3d, jnp.ones((_CH, F), jnp.float32), zeros, n_pad)
    degcol = (deg2[:n, 0] + deg2[n_pad:n_pad + n, 0]).reshape(n, 1)
    t0lo, t0hi, norm, norm2 = _tc_prep(degcol, x)
    p1lo, p1hi = _sc_prop(t0lo, t0hi, src4, dst3, zeros, n_pad)
    t1lo, t1hi = _tc_scale(p1lo, p1hi, norm2)
    p2lo, p2hi = _sc_prop(t1lo, t1hi, src4, dst3, zeros, n_pad)
    h1, t2lo, t2hi = _tc_mid_a(
        x, p1lo, p1hi, p2lo, p2hi, norm, W1, b1.reshape(1, hid), W2c)
    q1lo, q1hi = _sc_prop(t2lo, t2hi, src4, dst3, zeros, n_pad)
    v1, v2nlo, v2nhi = _tc_mid_b(h1, norm, W2ab)
    t3lo, t3hi = _tc_scale_add(q1lo, q1hi, norm2, v2nlo, v2nhi)
    q2lo, q2hi = _sc_prop(t3lo, t3hi, src4, dst3, zeros, n_pad)
    y = _tc_head(v1, q2lo, q2hi, norm, b2.reshape(1, out2), Wc,
                 bc.reshape(1, -1))
    return y


# speedup vs baseline: 1.0803x; 1.0798x over previous
import functools

import jax
import jax.numpy as jnp
from jax import lax
from jax.experimental import pallas as pl
from jax.experimental.pallas import tpu as pltpu
from jax.experimental.pallas import tpu_sc as plsc

NC = 2
NS = 16
LANES = 16
F = 128
BN = 2000


def _sc_mesh():
    return plsc.VectorSubcoreMesh(core_axis_name="c", subcore_axis_name="s")


_CH = 128
_CHP = 64


def _degree_body(n_pad, nchd, dst_hbm, ones_hbm, z_hbm, out_hbm, didx, onesv,
                 acc, sem0, sem1):
    c = lax.axis_index("c")
    s = lax.axis_index("s")
    wid = c * NS + s
    npw = n_pad // NS
    sems = (sem0, sem1)

    pltpu.sync_copy(ones_hbm, onesv)
    pltpu.sync_copy(dst_hbm.at[wid], didx)
    pltpu.sync_copy(z_hbm.at[pl.ds(s * npw, npw)], acc.at[pl.ds(s * npw, npw)])
    plsc.subcore_barrier()

    def gbody(g, carry):
        for b in range(2):
            k = g * 2 + b

            @pl.when(k >= 2)
            def _():
                pltpu.make_async_copy(onesv, acc.at[didx.at[k - 2]],
                                      sems[b]).wait()

            pltpu.async_copy(onesv, acc.at[didx.at[k]], sems[b], add=True)
        return carry

    lax.fori_loop(0, nchd // 2, gbody, 0)
    for b in range(2):
        pltpu.make_async_copy(onesv, acc.at[didx.at[nchd - 2 + b]],
                              sems[b]).wait()

    plsc.subcore_barrier()
    pltpu.sync_copy(acc.at[pl.ds(s * npw, npw)],
                    out_hbm.at[pl.ds(c * n_pad + s * npw, npw)])


def _sc_degree(dst3d, ones, zeros2d, n_pad):
    nchd = dst3d.shape[1]
    kfn = pl.kernel(
        functools.partial(_degree_body, n_pad, nchd),
        out_type=jax.ShapeDtypeStruct((NC * n_pad, F), jnp.float32),
        mesh=_sc_mesh(),
        scratch_types=[
            pltpu.VMEM((nchd, _CH), jnp.int32),
            pltpu.VMEM((_CH, F), jnp.float32),
            pltpu.VMEM_SHARED((n_pad, F), jnp.float32),
            pltpu.SemaphoreType.DMA,
            pltpu.SemaphoreType.DMA,
        ],
    )
    return kfn(dst3d, ones, zeros2d)


_RING = 3


def _prop_body(n_pad, nch, tlo_hbm, thi_hbm, src_hbm, dst_hbm, z_hbm, olo_hbm,
               ohi_hbm, didx, sidx, rows0, rows1, rows2, acc,
               gs0, gs1, gs2, ss0, ss1, ss2):
    c = lax.axis_index("c")
    s = lax.axis_index("s")
    rpt = n_pad // NS
    rows = (rows0, rows1, rows2)
    gsem = (gs0, gs1, gs2)
    ssem = (ss0, ss1, ss2)

    pltpu.sync_copy(dst_hbm.at[s], didx)
    pltpu.sync_copy(src_hbm.at[s], sidx)
    pltpu.sync_copy(z_hbm.at[pl.ds(s * rpt, rpt)], acc.at[pl.ds(s * rpt, rpt)])
    plsc.subcore_barrier()

    def svec(tbl, k):
        return tbl.at[k >> 1, pl.ds((k & 1) * _CHP, _CHP)]

    def edge_loop(table):
        def gbody(g, carry):
            for b in range(_RING):
                k = g * _RING + b
                bp = (b + _RING - 1) % _RING

                @pl.when(k >= _RING)
                def _():
                    pltpu.make_async_copy(rows[b], acc.at[svec(didx, k - _RING)],
                                          ssem[b]).wait()

                pltpu.async_copy(table.at[svec(sidx, k)], rows[b], gsem[b])

                @pl.when(k >= 1)
                def _():
                    pltpu.make_async_copy(table.at[svec(sidx, k - 1)], rows[bp],
                                          gsem[bp]).wait()
                    pltpu.async_copy(rows[bp], acc.at[svec(didx, k - 1)],
                                     ssem[bp], add=True)
            return carry

        lax.fori_loop(0, nch // _RING, gbody, 0)
        pltpu.make_async_copy(table.at[svec(sidx, nch - 1)], rows[_RING - 1],
                              gsem[_RING - 1]).wait()
        pltpu.async_copy(rows[_RING - 1], acc.at[svec(didx, nch - 1)],
                         ssem[_RING - 1], add=True)
        for b in range(_RING):
            pltpu.make_async_copy(rows[b], acc.at[svec(didx, b)],
                                  ssem[b]).wait()

    @pl.when(c == 0)
    def _():
        edge_loop(tlo_hbm)

    @pl.when(c == 1)
    def _():
        edge_loop(thi_hbm)

    plsc.subcore_barrier()

    @pl.when(c == 0)
    def _():
        pltpu.sync_copy(acc.at[pl.ds(s * rpt, rpt)],
                        olo_hbm.at[pl.ds(s * rpt, rpt)])

    @pl.when(c == 1)
    def _():
        pltpu.sync_copy(acc.at[pl.ds(s * rpt, rpt)],
                        ohi_hbm.at[pl.ds(s * rpt, rpt)])


def _sc_prop(tlo, thi, src4, dst3, zeros, n_pad):
    nch2 = src4.shape[1]
    kfn = pl.kernel(
        functools.partial(_prop_body, n_pad, 2 * nch2),
        out_type=(jax.ShapeDtypeStruct((n_pad, F), jnp.float32),
                  jax.ShapeDtypeStruct((n_pad, F), jnp.float32)),
        mesh=_sc_mesh(),
        scratch_types=[
            pltpu.VMEM((nch2, _CH), jnp.int32),
            pltpu.VMEM((nch2, _CH), jnp.int32),
            pltpu.VMEM((_CHP, F), jnp.float32),
            pltpu.VMEM((_CHP, F), jnp.float32),
            pltpu.VMEM((_CHP, F), jnp.float32),
            pltpu.VMEM_SHARED((n_pad, F), jnp.float32),
        ] + [pltpu.SemaphoreType.DMA] * 6,
    )
    return kfn(tlo, thi, src4, dst3, zeros)


def _prep_body(deg_ref, x_ref, t0lo_ref, t0hi_ref, norm_ref, norm2_ref):
    d = jnp.maximum(deg_ref[...], 1.0)
    norm = lax.rsqrt(d)
    xb = x_ref[...]
    t0lo_ref[...] = xb[:, :F] * norm
    t0hi_ref[...] = xb[:, F:] * norm
    norm_ref[...] = norm
    norm2_ref[...] = 1.0 / d


def _tc_prep(deg2, x):
    n = x.shape[0]
    g = n // BN
    return pl.pallas_call(
        _prep_body,
        grid=(g,),
        in_specs=[
            pl.BlockSpec((BN, 1), lambda i: (i, 0)),
            pl.BlockSpec((BN, 2 * F), lambda i: (i, 0)),
        ],
        out_specs=[
            pl.BlockSpec((BN, F), lambda i: (i, 0)),
            pl.BlockSpec((BN, F), lambda i: (i, 0)),
            pl.BlockSpec((BN, 1), lambda i: (i, 0)),
            pl.BlockSpec((BN, 1), lambda i: (i, 0)),
        ],
        out_shape=[
            jax.ShapeDtypeStruct((n, F), jnp.float32),
            jax.ShapeDtypeStruct((n, F), jnp.float32),
            jax.ShapeDtypeStruct((n, 1), jnp.float32),
            jax.ShapeDtypeStruct((n, 1), jnp.float32),
        ],
    )(deg2, x)


def _scale_body(alo_ref, ahi_ref, s_ref, olo_ref, ohi_ref):
    sb = s_ref[...]
    olo_ref[...] = alo_ref[...] * sb
    ohi_ref[...] = ahi_ref[...] * sb


def _tc_scale(alo, ahi, s):
    n = alo.shape[0]
    g = n // BN
    return pl.pallas_call(
        _scale_body,
        grid=(g,),
        in_specs=[
            pl.BlockSpec((BN, F), lambda i: (i, 0)),
            pl.BlockSpec((BN, F), lambda i: (i, 0)),
            pl.BlockSpec((BN, 1), lambda i: (i, 0)),
        ],
        out_specs=[
            pl.BlockSpec((BN, F), lambda i: (i, 0)),
            pl.BlockSpec((BN, F), lambda i: (i, 0)),
        ],
        out_shape=[
            jax.ShapeDtypeStruct((n, F), jnp.float32),
            jax.ShapeDtypeStruct((n, F), jnp.float32),
        ],
    )(alo, ahi, s)


def _scale_add_body(alo_ref, ahi_ref, s_ref, blo_ref, bhi_ref, olo_ref,
                    ohi_ref):
    sb = s_ref[...]
    olo_ref[...] = alo_ref[...] * sb + blo_ref[...]
    ohi_ref[...] = ahi_ref[...] * sb + bhi_ref[...]


def _tc_scale_add(alo, ahi, s, blo, bhi):
    n = alo.shape[0]
    g = n // BN
    return pl.pallas_call(
        _scale_add_body,
        grid=(g,),
        in_specs=[pl.BlockSpec((BN, F), lambda i: (i, 0)),
                  pl.BlockSpec((BN, F), lambda i: (i, 0)),
                  pl.BlockSpec((BN, 1), lambda i: (i, 0)),
                  pl.BlockSpec((BN, F), lambda i: (i, 0)),
                  pl.BlockSpec((BN, F), lambda i: (i, 0))],
        out_specs=[pl.BlockSpec((BN, F), lambda i: (i, 0)),
                   pl.BlockSpec((BN, F), lambda i: (i, 0))],
        out_shape=[jax.ShapeDtypeStruct((n, F), jnp.float32),
                   jax.ShapeDtypeStruct((n, F), jnp.float32)],
    )(alo, ahi, s, blo, bhi)


def _mid_a_body(x_ref, p1lo_ref, p1hi_ref, p2lo_ref, p2hi_ref, n_ref, W1_ref,
                b1_ref, W2c_ref, h1_ref, t2lo_ref, t2hi_ref):
    nb = n_ref[...]
    cat = jnp.concatenate(
        [x_ref[...],
         p1lo_ref[...] * nb, p1hi_ref[...] * nb,
         p2lo_ref[...] * nb, p2hi_ref[...] * nb], axis=1)
    h1 = jnp.dot(cat, W1_ref[...], preferred_element_type=jnp.float32)
    h1 = jnp.maximum(h1 + b1_ref[...], 0.0)
    h1_ref[...] = h1
    v = jnp.dot(h1, W2c_ref[...], preferred_element_type=jnp.float32)
    t2lo_ref[...] = v[:, :F] * nb
    t2hi_ref[...] = v[:, F:] * nb


def _tc_mid_a(x, p1lo, p1hi, p2lo, p2hi, norm, W1, b1, W2c):
    n = x.shape[0]
    g = n // BN
    in_dim = x.shape[1]
    hid = W1.shape[1]
    return pl.pallas_call(
        _mid_a_body,
        grid=(g,),
        in_specs=[
            pl.BlockSpec((BN, in_dim), lambda i: (i, 0)),
            pl.BlockSpec((BN, F), lambda i: (i, 0)),
            pl.BlockSpec((BN, F), lambda i: (i, 0)),
            pl.BlockSpec((BN, F), lambda i: (i, 0)),
            pl.BlockSpec((BN, F), lambda i: (i, 0)),
            pl.BlockSpec((BN, 1), lambda i: (i, 0)),
            pl.BlockSpec(W1.shape, lambda i: (0, 0)),
            pl.BlockSpec((1, hid), lambda i: (0, 0)),
            pl.BlockSpec(W2c.shape, lambda i: (0, 0)),
        ],
        out_specs=[
            pl.BlockSpec((BN, hid), lambda i: (i, 0)),
            pl.BlockSpec((BN, F), lambda i: (i, 0)),
            pl.BlockSpec((BN, F), lambda i: (i, 0)),
        ],
        out_shape=[
            jax.ShapeDtypeStruct((n, hid), jnp.float32),
            jax.ShapeDtypeStruct((n, F), jnp.float32),
            jax.ShapeDtypeStruct((n, F), jnp.float32),
        ],
    )(x, p1lo, p1hi, p2lo, p2hi, norm, W1, b1, W2c)


def _mid_b_body(h1_ref, n_ref, W2ab_ref, v1_ref, v2nlo_ref, v2nhi_ref):
    nb = n_ref[...]
    v = jnp.dot(h1_ref[...], W2ab_ref[...], preferred_element_type=jnp.float32)
    v1_ref[...] = v[:, :2 * F]
    v2nlo_ref[...] = v[:, 2 * F:3 * F] * nb
    v2nhi_ref[...] = v[:, 3 * F:4 * F] * nb


def _tc_mid_b(h1, norm, W2ab):
    n = h1.shape[0]
    g = n // BN
    hid = h1.shape[1]
    return pl.pallas_call(
        _mid_b_body,
        grid=(g,),
        in_specs=[
            pl.BlockSpec((BN, hid), lambda i: (i, 0)),
            pl.BlockSpec((BN, 1), lambda i: (i, 0)),
            pl.BlockSpec(W2ab.shape, lambda i: (0, 0)),
        ],
        out_specs=[
            pl.BlockSpec((BN, 2 * F), lambda i: (i, 0)),
            pl.BlockSpec((BN, F), lambda i: (i, 0)),
            pl.BlockSpec((BN, F), lambda i: (i, 0)),
        ],
        out_shape=[
            jax.ShapeDtypeStruct((n, 2 * F), jnp.float32),
            jax.ShapeDtypeStruct((n, F), jnp.float32),
            jax.ShapeDtypeStruct((n, F), jnp.float32),
        ],
    )(h1, norm, W2ab)


def _head_body(g, n, v1_ref, qlo_ref, qhi_ref, n_ref, b2_ref, Wc_ref, bc_ref,
               y_ref, acc_ref):
    i = pl.program_id(0)
    nb = n_ref[...]
    h2 = jnp.concatenate([qlo_ref[...], qhi_ref[...]], axis=1) * nb
    h2 = jnp.maximum(h2 + v1_ref[...] + b2_ref[...], 0.0)
    part = jnp.sum(h2, axis=0, keepdims=True)

    @pl.when(i == 0)
    def _():
        acc_ref[...] = part

    @pl.when(i > 0)
    def _():
        acc_ref[...] = acc_ref[...] + part

    @pl.when(i == g - 1)
    def _():
        hg = acc_ref[...] * (1.0 / n)
        y_ref[...] = (jnp.dot(hg, Wc_ref[...],
                              preferred_element_type=jnp.float32)
                      + bc_ref[...])


def _tc_head(v1, qlo, qhi, norm, b2, Wc, bc):
    n = v1.shape[0]
    g = n // BN
    ncls = Wc.shape[1]
    return pl.pallas_call(
        functools.partial(_head_body, g, float(n)),
        grid=(g,),
        in_specs=[
            pl.BlockSpec((BN, 2 * F), lambda i: (i, 0)),
            pl.BlockSpec((BN, F), lambda i: (i, 0)),
            pl.BlockSpec((BN, F), lambda i: (i, 0)),
            pl.BlockSpec((BN, 1), lambda i: (i, 0)),
            pl.BlockSpec((1, 2 * F), lambda i: (0, 0)),
            pl.BlockSpec(Wc.shape, lambda i: (0, 0)),
            pl.BlockSpec((1, ncls), lambda i: (0, 0)),
        ],
        out_specs=pl.BlockSpec((1, ncls), lambda i: (0, 0)),
        out_shape=jax.ShapeDtypeStruct((1, ncls), jnp.float32),
        scratch_shapes=[pltpu.VMEM((1, 2 * F), jnp.float32)],
    )(v1, qlo, qhi, norm, b2, Wc, bc)


def kernel(x, edge_index, W1, b1, W2, b2, Wc, bc):
    n, in_dim = x.shape
    e = edge_index.shape[1]
    hid = W1.shape[1]
    out2 = W2.shape[1]
    assert in_dim == 2 * F and n % BN == 0 and n % NS == 0

    src = edge_index[0]
    dst = edge_index[1]
    n_pad = ((n + NS * LANES - 1) // (NS * LANES)) * NS * LANES
    assert n_pad > n
    zeros = jnp.zeros((n_pad, F), jnp.float32)

    nch2 = -(-e // (NS * _CH))
    while (2 * nch2) % _RING:
        nch2 += 1
    ep = NS * nch2 * _CH
    ar = jnp.arange(ep - e, dtype=jnp.int32)
    src_p = jnp.concatenate([src, ar % n])
    dst_p = jnp.concatenate([dst, n + ar % (n_pad - n)])
    src4 = src_p.reshape(NS, nch2, _CH)
    dst3 = dst_p.reshape(NS, nch2, _CH)

    nchd = -(-e // (NC * NS * _CH))
    nchd = ((nchd + 1) // 2) * 2
    epd = NC * NS * nchd * _CH
    ard = jnp.arange(epd - e, dtype=jnp.int32)
    dst_pd = jnp.concatenate([dst, n + ard % (n_pad - n)])
    dst3d = dst_pd.reshape(NC * NS, nchd, _CH)

    W2c = W2[2 * hid:]
    W2ab = jnp.concatenate([W2[:hid], W2[hid:2 * hid]], axis=1)

    deg2 = _sc_degree(dst3d, jnp.ones((_CH, F), jnp.float32), zeros, n_pad)
    degcol = (deg2[:n, 0] + deg2[n_pad:n_pad + n, 0]).reshape(n, 1)
    t0lo, t0hi, norm, norm2 = _tc_prep(degcol, x)
    p1lo, p1hi = _sc_prop(t0lo, t0hi, src4, dst3, zeros, n_pad)
    t1lo, t1hi = _tc_scale(p1lo, p1hi, norm2)
    p2lo, p2hi = _sc_prop(t1lo, t1hi, src4, dst3, zeros, n_pad)
    h1, t2lo, t2hi = _tc_mid_a(
        x, p1lo, p1hi, p2lo, p2hi, norm, W1, b1.reshape(1, hid), W2c)
    q1lo, q1hi = _sc_prop(t2lo, t2hi, src4, dst3, zeros, n_pad)
    v1, v2nlo, v2nhi = _tc_mid_b(h1, norm, W2ab)
    t3lo, t3hi = _tc_scale_add(q1lo, q1hi, norm2, v2nlo, v2nhi)
    q2lo, q2hi = _sc_prop(t3lo, t3hi, src4, dst3, zeros, n_pad)
    y = _tc_head(v1, q2lo, q2hi, norm, b2.reshape(1, out2), Wc,
                 bc.reshape(1, -1))
    return y


# speedup vs baseline: 1.0839x; 1.0033x over previous
import functools

import jax
import jax.numpy as jnp
from jax import lax
from jax.experimental import pallas as pl
from jax.experimental.pallas import tpu as pltpu
from jax.experimental.pallas import tpu_sc as plsc

NC = 2
NS = 16
LANES = 16
F = 128
BN = 2000


def _sc_mesh():
    return plsc.VectorSubcoreMesh(core_axis_name="c", subcore_axis_name="s")


_CH = 128
_CHP = 64


_DW = 128


def _degree_body(n_pad, nchd, dst_hbm, ones_hbm, z_hbm, olo_hbm, ohi_hbm,
                 didx, onesv, acc, sem0, sem1):
    c = lax.axis_index("c")
    s = lax.axis_index("s")
    wid = c * NS + s
    npw = n_pad // NS
    sems = (sem0, sem1)

    pltpu.sync_copy(ones_hbm, onesv)
    pltpu.sync_copy(dst_hbm.at[wid], didx)
    pltpu.sync_copy(z_hbm.at[pl.ds(s * npw, npw)], acc.at[pl.ds(s * npw, npw)])
    plsc.subcore_barrier()

    def gbody(g, carry):
        for b in range(2):
            k = g * 2 + b

            @pl.when(k >= 2)
            def _():
                pltpu.make_async_copy(onesv, acc.at[didx.at[k - 2]],
                                      sems[b]).wait()

            pltpu.async_copy(onesv, acc.at[didx.at[k]], sems[b], add=True)
        return carry

    lax.fori_loop(0, nchd // 2, gbody, 0)
    for b in range(2):
        pltpu.make_async_copy(onesv, acc.at[didx.at[nchd - 2 + b]],
                              sems[b]).wait()

    plsc.subcore_barrier()

    @pl.when(c == 0)
    def _():
        pltpu.sync_copy(acc.at[pl.ds(s * npw, npw)],
                        olo_hbm.at[pl.ds(s * npw, npw)])

    @pl.when(c == 1)
    def _():
        pltpu.sync_copy(acc.at[pl.ds(s * npw, npw)],
                        ohi_hbm.at[pl.ds(s * npw, npw)])


def _sc_degree(dst3d, ones, zeros2d, n_pad):
    nchd = dst3d.shape[1]
    kfn = pl.kernel(
        functools.partial(_degree_body, n_pad, nchd),
        out_type=(jax.ShapeDtypeStruct((n_pad, _DW), jnp.float32),
                  jax.ShapeDtypeStruct((n_pad, _DW), jnp.float32)),
        mesh=_sc_mesh(),
        scratch_types=[
            pltpu.VMEM((nchd, _CH), jnp.int32),
            pltpu.VMEM((_CH, _DW), jnp.float32),
            pltpu.VMEM_SHARED((n_pad, _DW), jnp.float32),
            pltpu.SemaphoreType.DMA,
            pltpu.SemaphoreType.DMA,
        ],
    )
    return kfn(dst3d, ones, zeros2d)


_RING = 3


def _prop_body(n_pad, nch, tlo_hbm, thi_hbm, src_hbm, dst_hbm, z_hbm, olo_hbm,
               ohi_hbm, didx, sidx, rows0, rows1, rows2, acc,
               gs0, gs1, gs2, ss0, ss1, ss2):
    c = lax.axis_index("c")
    s = lax.axis_index("s")
    rpt = n_pad // NS
    rows = (rows0, rows1, rows2)
    gsem = (gs0, gs1, gs2)
    ssem = (ss0, ss1, ss2)

    pltpu.sync_copy(dst_hbm.at[s], didx)
    pltpu.sync_copy(src_hbm.at[s], sidx)
    pltpu.sync_copy(z_hbm.at[pl.ds(s * rpt, rpt)], acc.at[pl.ds(s * rpt, rpt)])
    plsc.subcore_barrier()

    def svec(tbl, k):
        return tbl.at[k >> 1, pl.ds((k & 1) * _CHP, _CHP)]

    def edge_loop(table):
        def gbody(g, carry):
            for b in range(_RING):
                k = g * _RING + b
                bp = (b + _RING - 1) % _RING

                @pl.when(k >= _RING)
                def _():
                    pltpu.make_async_copy(rows[b], acc.at[svec(didx, k - _RING)],
                                          ssem[b]).wait()

                pltpu.async_copy(table.at[svec(sidx, k)], rows[b], gsem[b])

                @pl.when(k >= 1)
                def _():
                    pltpu.make_async_copy(table.at[svec(sidx, k - 1)], rows[bp],
                                          gsem[bp]).wait()
                    pltpu.async_copy(rows[bp], acc.at[svec(didx, k - 1)],
                                     ssem[bp], add=True)
            return carry

        lax.fori_loop(0, nch // _RING, gbody, 0)
        pltpu.make_async_copy(table.at[svec(sidx, nch - 1)], rows[_RING - 1],
                              gsem[_RING - 1]).wait()
        pltpu.async_copy(rows[_RING - 1], acc.at[svec(didx, nch - 1)],
                         ssem[_RING - 1], add=True)
        for b in range(_RING):
            pltpu.make_async_copy(rows[b], acc.at[svec(didx, b)],
                                  ssem[b]).wait()

    @pl.when(c == 0)
    def _():
        edge_loop(tlo_hbm)

    @pl.when(c == 1)
    def _():
        edge_loop(thi_hbm)

    plsc.subcore_barrier()

    @pl.when(c == 0)
    def _():
        pltpu.sync_copy(acc.at[pl.ds(s * rpt, rpt)],
                        olo_hbm.at[pl.ds(s * rpt, rpt)])

    @pl.when(c == 1)
    def _():
        pltpu.sync_copy(acc.at[pl.ds(s * rpt, rpt)],
                        ohi_hbm.at[pl.ds(s * rpt, rpt)])


def _sc_prop(tlo, thi, src4, dst3, zeros, n_pad):
    nch2 = src4.shape[1]
    kfn = pl.kernel(
        functools.partial(_prop_body, n_pad, 2 * nch2),
        out_type=(jax.ShapeDtypeStruct((n_pad, F), jnp.float32),
                  jax.ShapeDtypeStruct((n_pad, F), jnp.float32)),
        mesh=_sc_mesh(),
        scratch_types=[
            pltpu.VMEM((nch2, _CH), jnp.int32),
            pltpu.VMEM((nch2, _CH), jnp.int32),
            pltpu.VMEM((_CHP, F), jnp.float32),
            pltpu.VMEM((_CHP, F), jnp.float32),
            pltpu.VMEM((_CHP, F), jnp.float32),
            pltpu.VMEM_SHARED((n_pad, F), jnp.float32),
        ] + [pltpu.SemaphoreType.DMA] * 6,
    )
    return kfn(tlo, thi, src4, dst3, zeros)


def _prep_body(dlo_ref, dhi_ref, x_ref, t0lo_ref, t0hi_ref, norm_ref,
               norm2_ref):
    d = jnp.maximum(dlo_ref[:, :1] + dhi_ref[:, :1], 1.0)
    norm = lax.rsqrt(d)
    xb = x_ref[...]
    t0lo_ref[...] = xb[:, :F] * norm
    t0hi_ref[...] = xb[:, F:] * norm
    norm_ref[...] = norm
    norm2_ref[...] = 1.0 / d


def _tc_prep(dlo, dhi, x):
    n = x.shape[0]
    g = n // BN
    return pl.pallas_call(
        _prep_body,
        grid=(g,),
        in_specs=[
            pl.BlockSpec((BN, _DW), lambda i: (i, 0)),
            pl.BlockSpec((BN, _DW), lambda i: (i, 0)),
            pl.BlockSpec((BN, 2 * F), lambda i: (i, 0)),
        ],
        out_specs=[
            pl.BlockSpec((BN, F), lambda i: (i, 0)),
            pl.BlockSpec((BN, F), lambda i: (i, 0)),
            pl.BlockSpec((BN, 1), lambda i: (i, 0)),
            pl.BlockSpec((BN, 1), lambda i: (i, 0)),
        ],
        out_shape=[
            jax.ShapeDtypeStruct((n, F), jnp.float32),
            jax.ShapeDtypeStruct((n, F), jnp.float32),
            jax.ShapeDtypeStruct((n, 1), jnp.float32),
            jax.ShapeDtypeStruct((n, 1), jnp.float32),
        ],
    )(dlo, dhi, x)


def _scale_body(alo_ref, ahi_ref, s_ref, olo_ref, ohi_ref):
    sb = s_ref[...]
    olo_ref[...] = alo_ref[...] * sb
    ohi_ref[...] = ahi_ref[...] * sb


def _tc_scale(alo, ahi, s):
    n = alo.shape[0]
    g = n // BN
    return pl.pallas_call(
        _scale_body,
        grid=(g,),
        in_specs=[
            pl.BlockSpec((BN, F), lambda i: (i, 0)),
            pl.BlockSpec((BN, F), lambda i: (i, 0)),
            pl.BlockSpec((BN, 1), lambda i: (i, 0)),
        ],
        out_specs=[
            pl.BlockSpec((BN, F), lambda i: (i, 0)),
            pl.BlockSpec((BN, F), lambda i: (i, 0)),
        ],
        out_shape=[
            jax.ShapeDtypeStruct((n, F), jnp.float32),
            jax.ShapeDtypeStruct((n, F), jnp.float32),
        ],
    )(alo, ahi, s)


def _scale_add_body(alo_ref, ahi_ref, s_ref, blo_ref, bhi_ref, olo_ref,
                    ohi_ref):
    sb = s_ref[...]
    olo_ref[...] = alo_ref[...] * sb + blo_ref[...]
    ohi_ref[...] = ahi_ref[...] * sb + bhi_ref[...]


def _tc_scale_add(alo, ahi, s, blo, bhi):
    n = alo.shape[0]
    g = n // BN
    return pl.pallas_call(
        _scale_add_body,
        grid=(g,),
        in_specs=[pl.BlockSpec((BN, F), lambda i: (i, 0)),
                  pl.BlockSpec((BN, F), lambda i: (i, 0)),
                  pl.BlockSpec((BN, 1), lambda i: (i, 0)),
                  pl.BlockSpec((BN, F), lambda i: (i, 0)),
                  pl.BlockSpec((BN, F), lambda i: (i, 0))],
        out_specs=[pl.BlockSpec((BN, F), lambda i: (i, 0)),
                   pl.BlockSpec((BN, F), lambda i: (i, 0))],
        out_shape=[jax.ShapeDtypeStruct((n, F), jnp.float32),
                   jax.ShapeDtypeStruct((n, F), jnp.float32)],
    )(alo, ahi, s, blo, bhi)


def _mid_a_body(x_ref, p1lo_ref, p1hi_ref, p2lo_ref, p2hi_ref, n_ref, W1_ref,
                b1_ref, W2c_ref, h1_ref, t2lo_ref, t2hi_ref):
    nb = n_ref[...]
    cat = jnp.concatenate(
        [x_ref[...],
         p1lo_ref[...] * nb, p1hi_ref[...] * nb,
         p2lo_ref[...] * nb, p2hi_ref[...] * nb], axis=1)
    h1 = jnp.dot(cat, W1_ref[...], preferred_element_type=jnp.float32)
    h1 = jnp.maximum(h1 + b1_ref[...], 0.0)
    h1_ref[...] = h1
    v = jnp.dot(h1, W2c_ref[...], preferred_element_type=jnp.float32)
    t2lo_ref[...] = v[:, :F] * nb
    t2hi_ref[...] = v[:, F:] * nb


def _tc_mid_a(x, p1lo, p1hi, p2lo, p2hi, norm, W1, b1, W2c):
    n = x.shape[0]
    g = n // BN
    in_dim = x.shape[1]
    hid = W1.shape[1]
    return pl.pallas_call(
        _mid_a_body,
        grid=(g,),
        in_specs=[
            pl.BlockSpec((BN, in_dim), lambda i: (i, 0)),
            pl.BlockSpec((BN, F), lambda i: (i, 0)),
            pl.BlockSpec((BN, F), lambda i: (i, 0)),
            pl.BlockSpec((BN, F), lambda i: (i, 0)),
            pl.BlockSpec((BN, F), lambda i: (i, 0)),
            pl.BlockSpec((BN, 1), lambda i: (i, 0)),
            pl.BlockSpec(W1.shape, lambda i: (0, 0)),
            pl.BlockSpec((1, hid), lambda i: (0, 0)),
            pl.BlockSpec(W2c.shape, lambda i: (0, 0)),
        ],
        out_specs=[
            pl.BlockSpec((BN, hid), lambda i: (i, 0)),
            pl.BlockSpec((BN, F), lambda i: (i, 0)),
            pl.BlockSpec((BN, F), lambda i: (i, 0)),
        ],
        out_shape=[
            jax.ShapeDtypeStruct((n, hid), jnp.float32),
            jax.ShapeDtypeStruct((n, F), jnp.float32),
            jax.ShapeDtypeStruct((n, F), jnp.float32),
        ],
    )(x, p1lo, p1hi, p2lo, p2hi, norm, W1, b1, W2c)


def _mid_b_body(h1_ref, n_ref, W2ab_ref, v1_ref, v2nlo_ref, v2nhi_ref):
    nb = n_ref[...]
    v = jnp.dot(h1_ref[...], W2ab_ref[...], preferred_element_type=jnp.float32)
    v1_ref[...] = v[:, :2 * F]
    v2nlo_ref[...] = v[:, 2 * F:3 * F] * nb
    v2nhi_ref[...] = v[:, 3 * F:4 * F] * nb


def _tc_mid_b(h1, norm, W2ab):
    n = h1.shape[0]
    g = n // BN
    hid = h1.shape[1]
    return pl.pallas_call(
        _mid_b_body,
        grid=(g,),
        in_specs=[
            pl.BlockSpec((BN, hid), lambda i: (i, 0)),
            pl.BlockSpec((BN, 1), lambda i: (i, 0)),
            pl.BlockSpec(W2ab.shape, lambda i: (0, 0)),
        ],
        out_specs=[
            pl.BlockSpec((BN, 2 * F), lambda i: (i, 0)),
            pl.BlockSpec((BN, F), lambda i: (i, 0)),
            pl.BlockSpec((BN, F), lambda i: (i, 0)),
        ],
        out_shape=[
            jax.ShapeDtypeStruct((n, 2 * F), jnp.float32),
            jax.ShapeDtypeStruct((n, F), jnp.float32),
            jax.ShapeDtypeStruct((n, F), jnp.float32),
        ],
    )(h1, norm, W2ab)


def _head_body(g, n, v1_ref, qlo_ref, qhi_ref, n_ref, b2_ref, Wc_ref, bc_ref,
               y_ref, acc_ref):
    i = pl.program_id(0)
    nb = n_ref[...]
    h2 = jnp.concatenate([qlo_ref[...], qhi_ref[...]], axis=1) * nb
    h2 = jnp.maximum(h2 + v1_ref[...] + b2_ref[...], 0.0)
    part = jnp.sum(h2, axis=0, keepdims=True)

    @pl.when(i == 0)
    def _():
        acc_ref[...] = part

    @pl.when(i > 0)
    def _():
        acc_ref[...] = acc_ref[...] + part

    @pl.when(i == g - 1)
    def _():
        hg = acc_ref[...] * (1.0 / n)
        y_ref[...] = (jnp.dot(hg, Wc_ref[...],
                              preferred_element_type=jnp.float32)
                      + bc_ref[...])


def _tc_head(v1, qlo, qhi, norm, b2, Wc, bc):
    n = v1.shape[0]
    g = n // BN
    ncls = Wc.shape[1]
    return pl.pallas_call(
        functools.partial(_head_body, g, float(n)),
        grid=(g,),
        in_specs=[
            pl.BlockSpec((BN, 2 * F), lambda i: (i, 0)),
            pl.BlockSpec((BN, F), lambda i: (i, 0)),
            pl.BlockSpec((BN, F), lambda i: (i, 0)),
            pl.BlockSpec((BN, 1), lambda i: (i, 0)),
            pl.BlockSpec((1, 2 * F), lambda i: (0, 0)),
            pl.BlockSpec(Wc.shape, lambda i: (0, 0)),
            pl.BlockSpec((1, ncls), lambda i: (0, 0)),
        ],
        out_specs=pl.BlockSpec((1, ncls), lambda i: (0, 0)),
        out_shape=jax.ShapeDtypeStruct((1, ncls), jnp.float32),
        scratch_shapes=[pltpu.VMEM((1, 2 * F), jnp.float32)],
    )(v1, qlo, qhi, norm, b2, Wc, bc)


def kernel(x, edge_index, W1, b1, W2, b2, Wc, bc):
    n, in_dim = x.shape
    e = edge_index.shape[1]
    hid = W1.shape[1]
    out2 = W2.shape[1]
    assert in_dim == 2 * F and n % BN == 0 and n % NS == 0

    src = edge_index[0]
    dst = edge_index[1]
    n_pad = ((n + NS * LANES - 1) // (NS * LANES)) * NS * LANES
    assert n_pad > n
    zeros = jnp.zeros((n_pad, F), jnp.float32)

    nch2 = -(-e // (NS * _CH))
    while (2 * nch2) % _RING:
        nch2 += 1
    ep = NS * nch2 * _CH
    ar = jnp.arange(ep - e, dtype=jnp.int32)
    src_p = jnp.concatenate([src, ar % n])
    dst_p = jnp.concatenate([dst, n + ar % (n_pad - n)])
    src4 = src_p.reshape(NS, nch2, _CH)
    dst3 = dst_p.reshape(NS, nch2, _CH)

    nchd = -(-e // (NC * NS * _CH))
    nchd = ((nchd + 1) // 2) * 2
    epd = NC * NS * nchd * _CH
    ard = jnp.arange(epd - e, dtype=jnp.int32)
    dst_pd = jnp.concatenate([dst, n + ard % (n_pad - n)])
    dst3d = dst_pd.reshape(NC * NS, nchd, _CH)

    W2c = W2[2 * hid:]
    W2ab = jnp.concatenate([W2[:hid], W2[hid:2 * hid]], axis=1)

    deglo, deghi = _sc_degree(dst3d, jnp.ones((_CH, _DW), jnp.float32),
                              jnp.zeros((n_pad, _DW), jnp.float32), n_pad)
    t0lo, t0hi, norm, norm2 = _tc_prep(deglo, deghi, x)
    p1lo, p1hi = _sc_prop(t0lo, t0hi, src4, dst3, zeros, n_pad)
    t1lo, t1hi = _tc_scale(p1lo, p1hi, norm2)
    p2lo, p2hi = _sc_prop(t1lo, t1hi, src4, dst3, zeros, n_pad)
    h1, t2lo, t2hi = _tc_mid_a(
        x, p1lo, p1hi, p2lo, p2hi, norm, W1, b1.reshape(1, hid), W2c)
    q1lo, q1hi = _sc_prop(t2lo, t2hi, src4, dst3, zeros, n_pad)
    v1, v2nlo, v2nhi = _tc_mid_b(h1, norm, W2ab)
    t3lo, t3hi = _tc_scale_add(q1lo, q1hi, norm2, v2nlo, v2nhi)
    q2lo, q2hi = _sc_prop(t3lo, t3hi, src4, dst3, zeros, n_pad)
    y = _tc_head(v1, q2lo, q2hi, norm, b2.reshape(1, out2), Wc,
                 bc.reshape(1, -1))
    return y
